# Initial kernel scaffold; baseline (speedup 1.0000x reference)
import jax
import jax.numpy as jnp
from jax.experimental import pallas as pl


def kernel(x, edge_index, edge_attr, W_ne, b_ne, g_ne, be_ne, W_ee, b_ee, g_ee, be_ee, Wa1, ba1, Wa2, ba2, W1, b1, g1, be1, W2, b2, eps):
    raise NotImplementedError("write your pallas kernel here")


# speedup vs baseline: 3.7291x; 3.7291x over previous
import functools

import jax
import jax.numpy as jnp
from jax import lax
from jax.experimental import pallas as pl
from jax.experimental.pallas import tpu as pltpu
from jax.experimental.pallas import tpu_sc as plsc


def _node_stage_kernel(x_ref, wne_ref, cne_ref, sne_ref, wa1t_ref, wa1b_ref,
                       ba1_ref, xt_ref, p_ref, q_ref):
    x = x_ref[...]
    h = jnp.dot(x, wne_ref[...], preferred_element_type=jnp.float32)
    xt = jnp.maximum(h * sne_ref[...] + cne_ref[...], 0.0)
    xt_ref[...] = xt
    p_ref[...] = jnp.dot(xt, wa1t_ref[...], preferred_element_type=jnp.float32) + ba1_ref[...]
    q_ref[...] = jnp.dot(xt, wa1b_ref[...], preferred_element_type=jnp.float32)


def _logit_stage_kernel(ea_ref, g_ref, wee_ref, cee_ref, see_ref, wa1t_ref,
                        wa2_ref, ba2_ref, ef_ref, l_ref):
    ea = ea_ref[...]
    h = jnp.dot(ea, wee_ref[...], preferred_element_type=jnp.float32)
    ef = jnp.maximum(h * see_ref[...] + cee_ref[...], 0.0)
    ef_ref[...] = ef
    r = jnp.dot(ef, wa1t_ref[...], preferred_element_type=jnp.float32)
    t = jnp.tanh(g_ref[...] + r)
    l = jnp.dot(t, wa2_ref[...], preferred_element_type=jnp.float32) + ba2_ref[...]
    l_ref[...] = jnp.where(l >= 0.0, l, 0.2 * l)


def _stats_kernel(l_ref, ms_ref):
    l = l_ref[...]
    m = jnp.max(l)
    s = jnp.sum(jnp.exp(l - m))
    ms_ref[...] = jnp.stack([jnp.full((16,), m, jnp.float32),
                             jnp.full((16,), 1.0 / s, jnp.float32)])


def _final_stage_kernel(xt_ref, a0_ref, a1_ref, eps_ref, w1_ref, c1_ref,
                        s1_ref, w2_ref, b2_ref, out_ref):
    y = (1.0 + eps_ref[0, 0]) * xt_ref[...] + (a0_ref[...] + a1_ref[...])
    h = jnp.dot(y, w1_ref[...], preferred_element_type=jnp.float32)
    h = jnp.maximum(h * s1_ref[...] + c1_ref[...], 0.0)
    out_ref[...] = jnp.dot(h, w2_ref[...], preferred_element_type=jnp.float32) + b2_ref[...]


def _full(shape):
    return pl.BlockSpec(shape, lambda *_: tuple(0 for _ in shape))


def _chunk_size(n, cap=128, step=8):
    k = (cap // step) * step
    while k > step and n % k:
        k -= step
    return k


def _make_gather_g(N, H, E):
    info = plsc.get_sparse_core_info()
    NC, NS = info.num_cores, info.num_subcores
    NW = NC * NS
    per_w = E // NW
    K = _chunk_size(per_w)
    nch = per_w // K
    UN = 5
    while nch % UN:
        UN -= 1
    AH = min(3, UN - 2)
    nit = nch // UN
    mesh = plsc.VectorSubcoreMesh(core_axis_name="c", subcore_axis_name="s")

    @functools.partial(
        pl.kernel,
        out_type=jax.ShapeDtypeStruct((E, H), jnp.float32),
        mesh=mesh,
        scratch_types=(
            [pltpu.VMEM((K,), jnp.int32)] * (2 * UN)
            + [pltpu.VMEM((K, H), jnp.float32)] * (2 * UN)
            + [pltpu.SemaphoreType.DMA] * (3 * UN)
        ),
    )
    def gather_g(p_hbm, q_hbm, row_hbm, col_hbm, g_hbm, *scr):
        idxr = scr[0:UN]
        idxc = scr[UN:2 * UN]
        bufp = scr[2 * UN:3 * UN]
        bufq = scr[3 * UN:4 * UN]
        semi = scr[4 * UN:5 * UN]
        semg = scr[5 * UN:6 * UN]
        semw = scr[6 * UN:7 * UN]
        wid = lax.axis_index("s") * NC + lax.axis_index("c")
        base_w = wid * per_w

        def ebase(ci):
            return pl.multiple_of(base_w + ci * K, 8)

        def idx_descs(ci, b):
            return (pltpu.make_async_copy(row_hbm.at[pl.ds(ebase(ci), K)],
                                          idxr[b], semi[b]),
                    pltpu.make_async_copy(col_hbm.at[pl.ds(ebase(ci), K)],
                                          idxc[b], semi[b]))

        def gat_descs(b):
            return (pltpu.make_async_copy(p_hbm.at[idxr[b]], bufp[b], semg[b]),
                    pltpu.make_async_copy(q_hbm.at[idxc[b]], bufq[b], semg[b]))

        def wr_desc(ci, b):
            return pltpu.make_async_copy(bufp[b], g_hbm.at[pl.ds(ebase(ci), K)],
                                         semw[b])

        for b in range(UN):
            for d in idx_descs(b, b):
                d.start()
        for b in range(AH):
            for d in idx_descs(b, b):
                d.wait()
            for d in gat_descs(b):
                d.start()

        def it(gi, carry):
            for b in range(UN):
                ci = gi * UN + b
                for d in gat_descs(b):
                    d.wait()

                @pl.when(ci + UN < nch)
                def _():
                    for d in idx_descs(ci + UN, b):
                        d.start()

                ba = (b + AH) % UN

                @pl.when(ci >= UN - AH)
                def _():
                    wr_desc(ci - (UN - AH), ba).wait()

                @pl.when(ci + AH < nch)
                def _():
                    for d in idx_descs(ci + AH, ba):
                        d.wait()
                    for d in gat_descs(ba):
                        d.start()

                def rowi(i, c2):
                    for j in range(H // 16):
                        sl = pl.ds(j * 16, 16)
                        bufp[b][i, sl] = bufp[b][i, sl] + bufq[b][i, sl]
                    return c2

                lax.fori_loop(0, K, rowi, 0)
                wr_desc(ci, b).start()

            return carry

        lax.fori_loop(0, nit, it, 0)
        for ci in range(nch - (UN - AH), nch):
            wr_desc(ci, ci % UN).wait()

    return gather_g


def _make_scatter(N, H, E):
    info = plsc.get_sparse_core_info()
    NC, NS = info.num_cores, info.num_subcores
    NW = NC * NS
    per_w = E // NW
    K = _chunk_size(per_w, cap=40)
    nch = per_w // K
    nit = (nch + 1) // 2
    nwr = NS
    while N % nwr or (N // nwr) % 8:
        nwr -= 1
    rows_per_tile = N // nwr
    ZR = _chunk_size(rows_per_tile)
    nz = rows_per_tile // ZR
    mesh = plsc.VectorSubcoreMesh(core_axis_name="c", subcore_axis_name="s")

    @functools.partial(
        pl.kernel,
        out_type=jax.ShapeDtypeStruct((NC, N, H), jnp.float32),
        mesh=mesh,
        scratch_types=(
            [pltpu.VMEM((K,), jnp.int32)] * 4
            + [pltpu.VMEM((K + 16,), jnp.float32)] * 2
            + [pltpu.VMEM((K, H), jnp.float32)] * 4
            + [pltpu.VMEM((2, 16), jnp.float32)]
            + [pltpu.VMEM((ZR, H), jnp.float32)]
            + [pltpu.VMEM_SHARED((N, H), jnp.float32)]
            + [pltpu.SemaphoreType.DMA] * 10
        ),
    )
    def scatter_msgs(xt_hbm, row_hbm, col_hbm, l_hbm, ms_hbm, ef_hbm, out_hbm, *scr):
        idxr = scr[0:2]
        idxc = scr[2:4]
        ubuf = scr[4:6]
        bufx = scr[6:8]
        bufe = scr[8:10]
        msbuf = scr[10]
        zbuf = scr[11]
        acc_sh = scr[12]
        sems = scr[13:]
        semi = sems[0:2]
        semr = sems[2:4]
        semg = sems[4:6]
        seme = sems[6:8]
        semw = sems[8:10]
        c = lax.axis_index("c")
        s = lax.axis_index("s")
        wid = s * NC + c
        base_w = wid * per_w

        def ebase(ci):
            return pl.multiple_of(base_w + ci * K, 8)

        def cu_descs(ci, b):
            return (pltpu.make_async_copy(col_hbm.at[pl.ds(ebase(ci), K)],
                                          idxc[b], semi[b]),
                    pltpu.make_async_copy(l_hbm.at[pl.ds(ebase(ci), K)],
                                          ubuf[b].at[pl.ds(0, K)], semi[b]))

        def r_desc(ci, b):
            return pltpu.make_async_copy(row_hbm.at[pl.ds(ebase(ci), K)],
                                         idxr[b], semr[b])

        def x_desc(b):
            return pltpu.make_async_copy(xt_hbm.at[idxc[b]], bufx[b], semg[b])

        def e_desc(ci, b):
            return pltpu.make_async_copy(ef_hbm.at[pl.ds(ebase(ci), K)],
                                         bufe[b], seme[b])

        def w_desc(b):
            return pltpu.make_async_copy(bufx[b], acc_sh.at[idxr[b]], semw[b])

        def zrow(i, carry):
            for j in range(H // 16):
                zbuf[i, pl.ds(j * 16, 16)] = jnp.zeros((16,), jnp.float32)
            return carry

        lax.fori_loop(0, ZR, zrow, 0)

        @pl.when(s < nwr)
        def _zero():
            def zcp(r, carry):
                pltpu.sync_copy(zbuf, acc_sh.at[pl.ds(pl.multiple_of(s * rows_per_tile + r * ZR, 8), ZR)])
                return carry
            lax.fori_loop(0, nz, zcp, 0)

        pltpu.sync_copy(ms_hbm, msbuf)
        mv = msbuf[0, :]
        sv = msbuf[1, :]

        for b in range(2):
            for d in cu_descs(b, b):
                d.start()
        r_desc(0, 0).start()
        for d in cu_descs(0, 0):
            d.wait()
        x_desc(0).start()
        e_desc(0, 0).start()

        plsc.subcore_barrier()

        def it(gi, carry):
            for b in range(2):
                ci = gi * 2 + b

                @pl.when(ci < nch)
                def _chunk():
                    x_desc(b).wait()
                    e_desc(ci, b).wait()

                    @pl.when(ci >= 1)
                    def _():
                        w_desc(1 - b).wait()

                    @pl.when(ci + 1 < nch)
                    def _():
                        r_desc(ci + 1, 1 - b).start()
                        for d in cu_descs(ci + 1, 1 - b):
                            d.wait()
                        x_desc(1 - b).start()
                        e_desc(ci + 1, 1 - b).start()

                    def group(g, c2):
                        lvec = ubuf[b][pl.ds(pl.multiple_of(g * 8, 8), 16)]
                        uvec = jnp.exp(lvec - mv) * sv
                        for r in range(8):
                            i = g * 8 + r
                            uv = lax.gather(
                                uvec, jnp.full((16, 1), r, jnp.int32),
                                lax.GatherDimensionNumbers(
                                    offset_dims=(), collapsed_slice_dims=(0,),
                                    start_index_map=(0,)),
                                (1,),
                                mode=lax.GatherScatterMode.PROMISE_IN_BOUNDS)
                            for j in range(H // 16):
                                sl = pl.ds(j * 16, 16)
                                bufx[b][i, sl] = (bufx[b][i, sl] + bufe[b][i, sl]) * uv
                        return c2

                    lax.fori_loop(0, K // 8, group, 0)

                    @pl.when(ci + 2 < nch)
                    def _():
                        for d in cu_descs(ci + 2, b):
                            d.start()

                    r_desc(ci, b).wait()
                    w_desc(b).start()

            return carry

        lax.fori_loop(0, nit, it, 0)
        w_desc((nch - 1) % 2).wait()
        plsc.subcore_barrier()

        @pl.when(s < nwr)
        def _writeback():
            rbase = pl.multiple_of(s * rows_per_tile, 8)
            pltpu.sync_copy(acc_sh.at[pl.ds(rbase, rows_per_tile)],
                            out_hbm.at[c, pl.ds(rbase, rows_per_tile)])

    return scatter_msgs


def kernel(x, edge_index, edge_attr, W_ne, b_ne, g_ne, be_ne, W_ee, b_ee,
           g_ee, be_ee, Wa1, ba1, Wa2, ba2, W1, b1, g1, be1, W2, b2, eps):
    N, D = x.shape
    E, ED = edge_attr.shape
    H = W_ne.shape[1]
    f32 = jnp.float32

    bn_inv = 1.0 / jnp.sqrt(jnp.float32(1.0 + 1e-5))
    s_ne = (g_ne * bn_inv).reshape(1, H)
    c_ne = (b_ne * g_ne * bn_inv + be_ne).reshape(1, H)
    s_ee = (g_ee * bn_inv).reshape(1, H)
    c_ee = (b_ee * g_ee * bn_inv + be_ee).reshape(1, H)
    s_1 = (g1 * bn_inv).reshape(1, 2 * H)
    c_1 = (b1 * g1 * bn_inv + be1).reshape(1, 2 * H)
    wa1_top = Wa1[:H]
    wa1_bot = Wa1[H:]
    row = edge_index[0]
    col = edge_index[1]

    xt, P, Q = pl.pallas_call(
        _node_stage_kernel,
        out_shape=[jax.ShapeDtypeStruct((N, H), f32)] * 3,
        in_specs=[_full((N, D)), _full((D, H)), _full((1, H)), _full((1, H)),
                  _full((H, H)), _full((H, H)), _full((1, H))],
        out_specs=[_full((N, H))] * 3,
    )(x, W_ne, c_ne, s_ne, wa1_top, wa1_bot, ba1.reshape(1, H))

    G = _make_gather_g(N, H, E)(P, Q, row, col)

    BE = 3200
    grid = (E // BE,)
    ef, l = pl.pallas_call(
        _logit_stage_kernel,
        grid=grid,
        out_shape=[jax.ShapeDtypeStruct((E, H), f32),
                   jax.ShapeDtypeStruct((E, 1), f32)],
        in_specs=[
            pl.BlockSpec((BE, ED), lambda i: (i, 0)),
            pl.BlockSpec((BE, H), lambda i: (i, 0)),
            _full((ED, H)), _full((1, H)), _full((1, H)), _full((H, H)),
            _full((H, 1)), _full((1, 1)),
        ],
        out_specs=[pl.BlockSpec((BE, H), lambda i: (i, 0)),
                   pl.BlockSpec((BE, 1), lambda i: (i, 0))],
    )(edge_attr, G, W_ee, c_ee, s_ee, wa1_top, Wa2, ba2.reshape(1, 1))

    LW = 512
    while E % LW:
        LW //= 2
    ms = pl.pallas_call(
        _stats_kernel,
        out_shape=jax.ShapeDtypeStruct((2, 16), f32),
        in_specs=[_full((E // LW, LW))],
        out_specs=_full((2, 16)),
    )(l.reshape(E // LW, LW))

    parts = _make_scatter(N, H, E)(xt, row, col, l.reshape(E), ms, ef)

    out = pl.pallas_call(
        _final_stage_kernel,
        out_shape=jax.ShapeDtypeStruct((N, H), f32),
        in_specs=[_full((N, H)), _full((N, H)), _full((N, H)),
                  pl.BlockSpec(memory_space=pltpu.SMEM),
                  _full((H, 2 * H)), _full((1, 2 * H)), _full((1, 2 * H)),
                  _full((2 * H, H)), _full((1, H))],
        out_specs=_full((N, H)),
    )(xt, parts[0], parts[1], eps.reshape(1, 1), W1, c_1, s_1, W2,
      b2.reshape(1, H))
    return out
